# baseline (speedup 1.0000x reference)
import functools

import jax
import jax.numpy as jnp
from jax import lax
from jax.experimental import pallas as pl
from jax.experimental.pallas import tpu as pltpu
from jax.experimental.pallas import tpu_sc as plsc

BATCH = 16384
EMBED = 32
GROUP = 8
CH = 32


def _make_gather():
  info = plsc.get_sparse_core_info()
  nc, ns = info.num_cores, info.num_subcores
  nw = nc * ns
  b_per_w = BATCH // nw
  n_ch = b_per_w // CH

  mesh = plsc.VectorSubcoreMesh(core_axis_name="c", subcore_axis_name="s")

  @functools.partial(
      pl.kernel,
      mesh=mesh,
      out_type=[
          jax.ShapeDtypeStruct((BATCH * GROUP, EMBED), jnp.float32),
          jax.ShapeDtypeStruct((BATCH * GROUP, EMBED), jnp.float32),
      ],
      scratch_types=[
          pltpu.VMEM((b_per_w,), jnp.int32),
          pltpu.VMEM((b_per_w,), jnp.int32),
          pltpu.VMEM((CH * GROUP, EMBED), jnp.float32),
          pltpu.VMEM((CH * GROUP, EMBED), jnp.float32),
          pltpu.SemaphoreType.DMA,
      ],
  )
  def gather(uidx_hbm, midx_hbm, utab_hbm, mtab_hbm, uout_hbm, mout_hbm,
             uidx_v, midx_v, ubuf_v, mbuf_v, sem):
    wid = lax.axis_index("s") * nc + lax.axis_index("c")
    base = wid * b_per_w
    pltpu.sync_copy(uidx_hbm.at[pl.ds(base, b_per_w)], uidx_v)
    pltpu.sync_copy(midx_hbm.at[pl.ds(base, b_per_w)], midx_v)

    for c in range(n_ch):
      def issue(g, carry):
        uvec = (uidx_v[pl.ds(c * CH + g * 16, 16)] >> 3) * GROUP
        mvec = (midx_v[pl.ds(c * CH + g * 16, 16)] >> 3) * GROUP
        for k in range(16):
          rr = pl.multiple_of(uvec[k], GROUP)
          pltpu.async_copy(utab_hbm.at[pl.ds(rr, GROUP)],
                           ubuf_v.at[pl.ds((g * 16 + k) * GROUP, GROUP)], sem)
          ss = pl.multiple_of(mvec[k], GROUP)
          pltpu.async_copy(mtab_hbm.at[pl.ds(ss, GROUP)],
                           mbuf_v.at[pl.ds((g * 16 + k) * GROUP, GROUP)], sem)
        return carry

      lax.fori_loop(0, CH // 16, issue, 0)

      def drain(i, carry):
        pltpu.make_async_copy(utab_hbm.at[pl.ds(0, GROUP)],
                              ubuf_v.at[pl.ds(0, GROUP)], sem).wait()
        pltpu.make_async_copy(mtab_hbm.at[pl.ds(0, GROUP)],
                              mbuf_v.at[pl.ds(0, GROUP)], sem).wait()
        return carry

      lax.fori_loop(0, CH, drain, 0)
      out_base = (base + c * CH) * GROUP
      pltpu.sync_copy(ubuf_v, uout_hbm.at[pl.ds(out_base, CH * GROUP)])
      pltpu.sync_copy(mbuf_v, mout_hbm.at[pl.ds(out_base, CH * GROUP)])

  return gather


def _mlp_body(u_ref, m_ref, uq_ref, mq_ref, w1_ref, b1_ref, w2_ref, b2_ref,
              w3_ref, b3_ref, o_ref):
  uq = uq_ref[...]
  mq = mq_ref[...]
  u = jnp.zeros((u_ref.shape[0], EMBED), jnp.float32)
  m = jnp.zeros_like(u)
  for q in range(GROUP):
    u = u + jnp.where(uq == q, 1.0, 0.0) * u_ref[:, q * EMBED:(q + 1) * EMBED]
    m = m + jnp.where(mq == q, 1.0, 0.0) * m_ref[:, q * EMBED:(q + 1) * EMBED]
  h1 = jnp.dot(u, w1_ref[0:EMBED, :], preferred_element_type=jnp.float32)
  h1 = h1 + jnp.dot(m, w1_ref[EMBED:2 * EMBED, :],
                    preferred_element_type=jnp.float32)
  h1 = jnp.maximum(h1 + b1_ref[...], 0.0)
  h2 = jnp.dot(h1, w2_ref[...], preferred_element_type=jnp.float32)
  h2 = jnp.maximum(h2 + b2_ref[...], 0.0)
  o_ref[...] = jnp.sum(h2 * w3_ref[...], axis=1, keepdims=True) + b3_ref[...]


def _mlp_call(u_rows, m_rows, uq, mq, W1, b1, W2, b2, W3, b3):
  bb = 1024
  grid = (BATCH // bb,)
  w = GROUP * EMBED
  return pl.pallas_call(
      _mlp_body,
      grid=grid,
      in_specs=[
          pl.BlockSpec((bb, w), lambda i: (i, 0)),
          pl.BlockSpec((bb, w), lambda i: (i, 0)),
          pl.BlockSpec((bb, 1), lambda i: (i, 0)),
          pl.BlockSpec((bb, 1), lambda i: (i, 0)),
          pl.BlockSpec((2 * EMBED, 128), lambda i: (0, 0)),
          pl.BlockSpec((1, 128), lambda i: (0, 0)),
          pl.BlockSpec((128, 64), lambda i: (0, 0)),
          pl.BlockSpec((1, 64), lambda i: (0, 0)),
          pl.BlockSpec((1, 64), lambda i: (0, 0)),
          pl.BlockSpec((1, 1), lambda i: (0, 0)),
      ],
      out_specs=pl.BlockSpec((bb, 1), lambda i: (i, 0)),
      out_shape=jax.ShapeDtypeStruct((BATCH, 1), jnp.float32),
  )(u_rows, m_rows, uq, mq, W1, b1.reshape(1, 128), W2, b2.reshape(1, 64),
    W3.reshape(1, 64), b3.reshape(1, 1))


def kernel(user_input, movie_input, user_table, movie_table,
           W1, b1, W2, b2, W3, b3):
  gather = _make_gather()
  u8, m8 = gather(user_input, movie_input, user_table, movie_table)
  u_rows = u8.reshape(BATCH, GROUP * EMBED)
  m_rows = m8.reshape(BATCH, GROUP * EMBED)
  uq = (user_input & 7).reshape(BATCH, 1)
  mq = (movie_input & 7).reshape(BATCH, 1)
  return _mlp_call(u_rows, m_rows, uq, mq, W1, b1, W2, b2, W3, b3)


# speedup vs baseline: 1.1082x; 1.1082x over previous
import functools

import jax
import jax.numpy as jnp
from jax import lax
from jax.experimental import pallas as pl
from jax.experimental.pallas import tpu as pltpu
from jax.experimental.pallas import tpu_sc as plsc

BATCH = 16384
EMBED = 32
GROUP = 8
CH = 32


def _make_gather():
  info = plsc.get_sparse_core_info()
  nc, ns = info.num_cores, info.num_subcores
  nw = nc * ns
  b_per_w = BATCH // nw
  n_ch = b_per_w // CH

  mesh = plsc.VectorSubcoreMesh(core_axis_name="c", subcore_axis_name="s")

  @functools.partial(
      pl.kernel,
      mesh=mesh,
      out_type=[
          jax.ShapeDtypeStruct((BATCH * GROUP, EMBED), jnp.float32),
          jax.ShapeDtypeStruct((BATCH * GROUP, EMBED), jnp.float32),
      ],
      scratch_types=[
          pltpu.VMEM((b_per_w,), jnp.int32),
          pltpu.VMEM((b_per_w,), jnp.int32),
          pltpu.VMEM((CH * GROUP, EMBED), jnp.float32),
          pltpu.VMEM((CH * GROUP, EMBED), jnp.float32),
          pltpu.SemaphoreType.DMA,
      ],
  )
  def gather(uidx_hbm, midx_hbm, utab_hbm, mtab_hbm, uout_hbm, mout_hbm,
             uidx_v, midx_v, ubuf_v, mbuf_v, sem):
    wid = lax.axis_index("s") * nc + lax.axis_index("c")
    base = wid * b_per_w
    pltpu.sync_copy(uidx_hbm.at[pl.ds(base, b_per_w)], uidx_v)
    pltpu.sync_copy(midx_hbm.at[pl.ds(base, b_per_w)], midx_v)

    for c in range(n_ch):
      def issue(g, carry):
        uvec = (uidx_v[pl.ds(c * CH + g * 16, 16)] >> 3) * GROUP
        mvec = (midx_v[pl.ds(c * CH + g * 16, 16)] >> 3) * GROUP
        for k in range(16):
          rr = pl.multiple_of(uvec[k], GROUP)
          pltpu.async_copy(utab_hbm.at[pl.ds(rr, GROUP)],
                           ubuf_v.at[pl.ds((g * 16 + k) * GROUP, GROUP)], sem)
          ss = pl.multiple_of(mvec[k], GROUP)
          pltpu.async_copy(mtab_hbm.at[pl.ds(ss, GROUP)],
                           mbuf_v.at[pl.ds((g * 16 + k) * GROUP, GROUP)], sem)
        return carry

      lax.fori_loop(0, CH // 16, issue, 0)

      pltpu.make_async_copy(uout_hbm.at[pl.ds(0, CH * GROUP)], ubuf_v,
                            sem).wait()
      pltpu.make_async_copy(mout_hbm.at[pl.ds(0, CH * GROUP)], mbuf_v,
                            sem).wait()
      out_base = (base + c * CH) * GROUP
      pltpu.sync_copy(ubuf_v, uout_hbm.at[pl.ds(out_base, CH * GROUP)])
      pltpu.sync_copy(mbuf_v, mout_hbm.at[pl.ds(out_base, CH * GROUP)])

  return gather


def _mlp_body(u_ref, m_ref, uq_ref, mq_ref, w1_ref, b1_ref, w2_ref, b2_ref,
              w3_ref, b3_ref, o_ref):
  bb = uq_ref.shape[0]
  uq = uq_ref[...]
  mq = mq_ref[...]
  u3 = u_ref[...].reshape(bb, GROUP, EMBED)
  m3 = m_ref[...].reshape(bb, GROUP, EMBED)
  u = jnp.zeros((bb, EMBED), jnp.float32)
  m = jnp.zeros_like(u)
  for q in range(GROUP):
    u = u + jnp.where(uq == q, 1.0, 0.0) * u3[:, q, :]
    m = m + jnp.where(mq == q, 1.0, 0.0) * m3[:, q, :]
  h1 = jnp.dot(u, w1_ref[0:EMBED, :], preferred_element_type=jnp.float32)
  h1 = h1 + jnp.dot(m, w1_ref[EMBED:2 * EMBED, :],
                    preferred_element_type=jnp.float32)
  h1 = jnp.maximum(h1 + b1_ref[...], 0.0)
  h2 = jnp.dot(h1, w2_ref[...], preferred_element_type=jnp.float32)
  h2 = jnp.maximum(h2 + b2_ref[...], 0.0)
  o_ref[...] = jnp.sum(h2 * w3_ref[...], axis=1, keepdims=True) + b3_ref[...]


def _mlp_call(u_rows, m_rows, uq, mq, W1, b1, W2, b2, W3, b3):
  bb = 1024
  grid = (BATCH // bb,)
  return pl.pallas_call(
      _mlp_body,
      grid=grid,
      in_specs=[
          pl.BlockSpec((bb * GROUP, EMBED), lambda i: (i, 0)),
          pl.BlockSpec((bb * GROUP, EMBED), lambda i: (i, 0)),
          pl.BlockSpec((bb, 1), lambda i: (i, 0)),
          pl.BlockSpec((bb, 1), lambda i: (i, 0)),
          pl.BlockSpec((2 * EMBED, 128), lambda i: (0, 0)),
          pl.BlockSpec((1, 128), lambda i: (0, 0)),
          pl.BlockSpec((128, 64), lambda i: (0, 0)),
          pl.BlockSpec((1, 64), lambda i: (0, 0)),
          pl.BlockSpec((1, 64), lambda i: (0, 0)),
          pl.BlockSpec((1, 1), lambda i: (0, 0)),
      ],
      out_specs=pl.BlockSpec((bb, 1), lambda i: (i, 0)),
      out_shape=jax.ShapeDtypeStruct((BATCH, 1), jnp.float32),
  )(u_rows, m_rows, uq, mq, W1, b1.reshape(1, 128), W2, b2.reshape(1, 64),
    W3.reshape(1, 64), b3.reshape(1, 1))


def kernel(user_input, movie_input, user_table, movie_table,
           W1, b1, W2, b2, W3, b3):
  gather = _make_gather()
  u_rows, m_rows = gather(user_input, movie_input, user_table, movie_table)
  uq = (user_input & 7).reshape(BATCH, 1)
  mq = (movie_input & 7).reshape(BATCH, 1)
  return _mlp_call(u_rows, m_rows, uq, mq, W1, b1, W2, b2, W3, b3)
